# Initial kernel scaffold; baseline (speedup 1.0000x reference)
import jax
import jax.numpy as jnp
from jax.experimental import pallas as pl

TILE = 16


def _fused_body(atoms_ref, bonds_ref, w1_ref, b1_ref, w2_ref, b2_ref,
                wo_ref, bo_ref, edges_ref, out_ref):
    t, n, af = atoms_ref.shape
    d = edges_ref.shape[-1]
    hid = w1_ref.shape[-1]
    bf = bonds_ref.shape[-1]

    atoms = atoms_ref[...]
    sb = jnp.sum(bonds_ref[...], axis=2)
    sb2 = sb.reshape(t * n, bf)
    e = edges_ref[...]

    def nbr(x, k):
        f = x.shape[-1]
        idx = jnp.broadcast_to(e[:, :, k][..., None], (t, n, f))
        return jnp.take_along_axis(x, idx, axis=1, mode="promise_in_bounds")

    def gsum(x):
        s = x
        for k in range(d):
            s = s + nbr(x, k)
        return s

    def gmax(x):
        m = x
        for k in range(d):
            m = jnp.maximum(m, nbr(x, k))
        return m

    def conv(x, w_ref, b_ref):
        f = x.shape[-1]
        s = gsum(x).reshape(t * n, f)
        z = (jnp.dot(s, w_ref[:f, :], preferred_element_type=jnp.float32)
             + jnp.dot(sb2, w_ref[f:, :], preferred_element_type=jnp.float32)
             + b_ref[...])
        return jnp.maximum(z, 0.0).reshape(t, n, hid)

    h = conv(atoms, w1_ref, b1_ref)
    h = gmax(h)
    h = conv(h, w2_ref, b2_ref)
    h = gmax(h)

    z = (jnp.dot(h.reshape(t * n, hid), wo_ref[:hid, :],
                 preferred_element_type=jnp.float32)
         + jnp.dot(sb2, wo_ref[hid:, :], preferred_element_type=jnp.float32)
         + bo_ref[...])
    z = z - jnp.max(z, axis=-1, keepdims=True)
    p = jnp.exp(z)
    p = p / jnp.sum(p, axis=-1, keepdims=True)
    out_ref[...] = jnp.sum(p.reshape(t, n, hid), axis=1)


def kernel(atoms, bonds, W1, b1, W2, b2, Wo, bo, edges):
    b, n, af = atoms.shape
    d = edges.shape[-1]
    hid = W1.shape[-1]
    t = TILE
    e32 = edges.astype(jnp.int32)
    b1r = b1.reshape(1, hid)
    b2r = b2.reshape(1, hid)
    bor = bo.reshape(1, hid)

    grid = (b // t,)
    out = pl.pallas_call(
        _fused_body,
        grid=grid,
        in_specs=[
            pl.BlockSpec((t, n, af), lambda i: (i, 0, 0)),
            pl.BlockSpec((t, n, d, bonds.shape[-1]), lambda i: (i, 0, 0, 0)),
            pl.BlockSpec(W1.shape, lambda i: (0, 0)),
            pl.BlockSpec((1, hid), lambda i: (0, 0)),
            pl.BlockSpec(W2.shape, lambda i: (0, 0)),
            pl.BlockSpec((1, hid), lambda i: (0, 0)),
            pl.BlockSpec(Wo.shape, lambda i: (0, 0)),
            pl.BlockSpec((1, hid), lambda i: (0, 0)),
            pl.BlockSpec((t, n, d), lambda i: (i, 0, 0)),
        ],
        out_specs=pl.BlockSpec((t, hid), lambda i: (i, 0)),
        out_shape=jax.ShapeDtypeStruct((b, hid), jnp.float32),
    )(atoms, bonds, W1, b1r, W2, b2r, Wo, bor, e32)
    return out


# speedup vs baseline: 28.3865x; 28.3865x over previous
import jax
import jax.numpy as jnp
from jax.experimental import pallas as pl

TILE = 16


def _fused_body(atoms_ref, bonds_ref, w1_ref, b1_ref, w2_ref, b2_ref,
                wo_ref, bo_ref, edges_ref, out_ref):
    t, n, af = atoms_ref.shape
    d = edges_ref.shape[-1]
    hid = w1_ref.shape[-1]
    bf = bonds_ref.shape[-1]

    atoms = atoms_ref[...]
    sb = jnp.sum(bonds_ref[...], axis=2)
    sb2 = sb.reshape(t * n, bf)
    e = edges_ref[...]

    def nbr_fm(x_fm, k):
        f = x_fm.shape[1]
        idx = jnp.broadcast_to(e[:, :, k][:, None, :], (t, f, n))
        return jnp.take_along_axis(x_fm, idx, axis=2,
                                   mode="promise_in_bounds")

    def gsum_fm(x_fm):
        s = x_fm
        for k in range(d):
            s = s + nbr_fm(x_fm, k)
        return s

    def gmax_fm(x_fm):
        m = x_fm
        for k in range(d):
            m = jnp.maximum(m, nbr_fm(x_fm, k))
        return m

    def dense(s_am, w_ref, b_ref):
        f = s_am.shape[-1]
        z = (jnp.dot(s_am, w_ref[:f, :], preferred_element_type=jnp.float32)
             + jnp.dot(sb2, w_ref[f:, :], preferred_element_type=jnp.float32)
             + b_ref[...])
        return z

    a_fm = jnp.swapaxes(atoms, 1, 2)
    s1 = jnp.swapaxes(gsum_fm(a_fm), 1, 2).reshape(t * n, af)
    h1 = jnp.maximum(dense(s1, w1_ref, b1_ref), 0.0)
    h1_fm = jnp.swapaxes(h1.reshape(t, n, hid), 1, 2)
    m1_fm = gmax_fm(h1_fm)
    s2 = jnp.swapaxes(gsum_fm(m1_fm), 1, 2).reshape(t * n, hid)
    h2 = jnp.maximum(dense(s2, w2_ref, b2_ref), 0.0)
    h2_fm = jnp.swapaxes(h2.reshape(t, n, hid), 1, 2)
    m2 = jnp.swapaxes(gmax_fm(h2_fm), 1, 2).reshape(t * n, hid)

    z = dense(m2, wo_ref, bo_ref)
    z = z - jnp.max(z, axis=-1, keepdims=True)
    p = jnp.exp(z)
    p = p / jnp.sum(p, axis=-1, keepdims=True)
    out_ref[...] = jnp.sum(p.reshape(t, n, hid), axis=1)


def kernel(atoms, bonds, W1, b1, W2, b2, Wo, bo, edges):
    b, n, af = atoms.shape
    d = edges.shape[-1]
    hid = W1.shape[-1]
    t = TILE
    e32 = edges.astype(jnp.int32)
    b1r = b1.reshape(1, hid)
    b2r = b2.reshape(1, hid)
    bor = bo.reshape(1, hid)

    grid = (b // t,)
    out = pl.pallas_call(
        _fused_body,
        grid=grid,
        in_specs=[
            pl.BlockSpec((t, n, af), lambda i: (i, 0, 0)),
            pl.BlockSpec((t, n, d, bonds.shape[-1]), lambda i: (i, 0, 0, 0)),
            pl.BlockSpec(W1.shape, lambda i: (0, 0)),
            pl.BlockSpec((1, hid), lambda i: (0, 0)),
            pl.BlockSpec(W2.shape, lambda i: (0, 0)),
            pl.BlockSpec((1, hid), lambda i: (0, 0)),
            pl.BlockSpec(Wo.shape, lambda i: (0, 0)),
            pl.BlockSpec((1, hid), lambda i: (0, 0)),
            pl.BlockSpec((t, n, d), lambda i: (i, 0, 0)),
        ],
        out_specs=pl.BlockSpec((t, hid), lambda i: (i, 0)),
        out_shape=jax.ShapeDtypeStruct((b, hid), jnp.float32),
    )(atoms, bonds, W1, b1r, W2, b2r, Wo, bor, e32)
    return out


# speedup vs baseline: 54.3466x; 1.9145x over previous
import jax
import jax.numpy as jnp
from jax.experimental import pallas as pl

TILE = 16


def _fused_body(atoms_ref, bonds_ref, w1a_ref, w1b_ref, b1_ref,
                w2a_ref, w2b_ref, b2_ref, wo_ref, wob_ref, bo_ref,
                edges_ref, out_ref):
    t, n, af = atoms_ref.shape
    d = edges_ref.shape[-1]
    hid = w1a_ref.shape[-1]
    t2 = t // 2
    nn = 2 * n

    bonds_flat = bonds_ref[...].reshape(t * n, bonds_ref.shape[-1])
    e2 = edges_ref[...].reshape(t2, nn, d)
    lane = jax.lax.broadcasted_iota(jnp.int32, (t2, nn), 1)
    off = jnp.where(lane >= n, n, 0)
    idx = [jnp.broadcast_to((e2[:, :, k] + off)[:, None, :], (t2, hid, nn))
           for k in range(d)]

    def gsum_fm(x_fm):
        f = x_fm.shape[1]
        s = x_fm
        for k in range(d):
            s = s + jnp.take_along_axis(x_fm, idx[k][:, :f, :], axis=2,
                                        mode="promise_in_bounds")
        return s

    def gmax_fm(x_fm):
        f = x_fm.shape[1]
        m = x_fm
        for k in range(d):
            m = jnp.maximum(m, jnp.take_along_axis(x_fm, idx[k][:, :f, :],
                                                   axis=2,
                                                   mode="promise_in_bounds"))
        return m

    def to_am(x_fm):
        return jnp.swapaxes(x_fm, 1, 2).reshape(t * n, x_fm.shape[1])

    def to_fm(x_am):
        return jnp.swapaxes(x_am.reshape(t2, nn, x_am.shape[-1]), 1, 2)

    def dense(s_am, wa_ref, wb_ref, b_ref):
        return (jnp.dot(s_am, wa_ref[...], preferred_element_type=jnp.float32)
                + jnp.dot(bonds_flat, wb_ref[...],
                          preferred_element_type=jnp.float32)
                + b_ref[...])

    a_fm = to_fm(atoms_ref[...].reshape(t * n, af))
    s1 = to_am(gsum_fm(a_fm))
    h1 = jnp.maximum(dense(s1, w1a_ref, w1b_ref, b1_ref), 0.0)
    m1_fm = gmax_fm(to_fm(h1))
    s2 = to_am(gsum_fm(m1_fm))
    h2 = jnp.maximum(dense(s2, w2a_ref, w2b_ref, b2_ref), 0.0)
    m2 = to_am(gmax_fm(to_fm(h2)))
    z = dense(m2, wo_ref, wob_ref, bo_ref)
    z = z - jnp.max(z, axis=-1, keepdims=True)
    p = jnp.exp(z)
    p = p / jnp.sum(p, axis=-1, keepdims=True)
    out_ref[...] = jnp.sum(p.reshape(t, n, hid), axis=1)


def kernel(atoms, bonds, W1, b1, W2, b2, Wo, bo, edges):
    b, n, af = atoms.shape
    d = edges.shape[-1]
    bf = bonds.shape[-1]
    hid = W1.shape[-1]
    t = TILE
    e32 = edges.astype(jnp.int32)
    bonds_flat = bonds.reshape(b, n, d * bf)
    w1a, w1b = W1[:af], jnp.tile(W1[af:], (d, 1))
    w2a, w2b = W2[:hid], jnp.tile(W2[hid:], (d, 1))
    woa, wob = Wo[:hid], jnp.tile(Wo[hid:], (d, 1))
    b1r = b1.reshape(1, hid)
    b2r = b2.reshape(1, hid)
    bor = bo.reshape(1, hid)

    grid = (b // t,)
    full = lambda s: pl.BlockSpec(s, lambda i: tuple(0 for _ in s))
    out = pl.pallas_call(
        _fused_body,
        grid=grid,
        in_specs=[
            pl.BlockSpec((t, n, af), lambda i: (i, 0, 0)),
            pl.BlockSpec((t, n, d * bf), lambda i: (i, 0, 0)),
            full(w1a.shape), full(w1b.shape), full(b1r.shape),
            full(w2a.shape), full(w2b.shape), full(b2r.shape),
            full(woa.shape), full(wob.shape), full(bor.shape),
            pl.BlockSpec((t, n, d), lambda i: (i, 0, 0)),
        ],
        out_specs=pl.BlockSpec((t, hid), lambda i: (i, 0)),
        out_shape=jax.ShapeDtypeStruct((b, hid), jnp.float32),
    )(atoms, bonds_flat, w1a, w1b, b1r, w2a, w2b, b2r, woa, wob, bor, e32)
    return out


# speedup vs baseline: 61.3688x; 1.1292x over previous
import jax
import jax.numpy as jnp
from jax.experimental import pallas as pl

TILE = 64


def _fused_body(atoms_ref, bonds_ref, w1a_ref, w1b_ref, b1_ref,
                w2a_ref, w2b_ref, b2_ref, wo_ref, wob_ref, bo_ref,
                edges_ref, out_ref):
    t, n, af = atoms_ref.shape
    d = edges_ref.shape[-1]
    hid = w1a_ref.shape[-1]
    t2 = t // 2
    nn = 2 * n

    bonds_flat = bonds_ref[...].reshape(t * n, bonds_ref.shape[-1])
    e2 = edges_ref[...].reshape(t2, nn, d)
    lane = jax.lax.broadcasted_iota(jnp.int32, (t2, nn), 1)
    off = jnp.where(lane >= n, n, 0)
    idx = [jnp.broadcast_to((e2[:, :, k] + off)[:, None, :], (t2, hid, nn))
           for k in range(d)]

    def gsum_fm(x_fm):
        f = x_fm.shape[1]
        s = x_fm
        for k in range(d):
            s = s + jnp.take_along_axis(x_fm, idx[k][:, :f, :], axis=2,
                                        mode="promise_in_bounds")
        return s

    def gmax_fm(x_fm):
        f = x_fm.shape[1]
        m = x_fm
        for k in range(d):
            m = jnp.maximum(m, jnp.take_along_axis(x_fm, idx[k][:, :f, :],
                                                   axis=2,
                                                   mode="promise_in_bounds"))
        return m

    def to_am(x_fm):
        return jnp.swapaxes(x_fm, 1, 2).reshape(t * n, x_fm.shape[1])

    def to_fm(x_am):
        return jnp.swapaxes(x_am.reshape(t2, nn, x_am.shape[-1]), 1, 2)

    def dense(s_am, wa_ref, wb_ref, b_ref):
        return (jnp.dot(s_am, wa_ref[...], preferred_element_type=jnp.float32)
                + jnp.dot(bonds_flat, wb_ref[...],
                          preferred_element_type=jnp.float32)
                + b_ref[...])

    a_fm = to_fm(atoms_ref[...].reshape(t * n, af))
    s1 = to_am(gsum_fm(a_fm))
    h1 = jnp.maximum(dense(s1, w1a_ref, w1b_ref, b1_ref), 0.0)
    m1_fm = gmax_fm(to_fm(h1))
    s2 = to_am(gsum_fm(m1_fm))
    h2 = jnp.maximum(dense(s2, w2a_ref, w2b_ref, b2_ref), 0.0)
    m2 = to_am(gmax_fm(to_fm(h2)))
    z = dense(m2, wo_ref, wob_ref, bo_ref)
    z = z - jnp.max(z, axis=-1, keepdims=True)
    p = jnp.exp(z)
    p = p / jnp.sum(p, axis=-1, keepdims=True)
    out_ref[...] = jnp.sum(p.reshape(t, n, hid), axis=1)


def kernel(atoms, bonds, W1, b1, W2, b2, Wo, bo, edges):
    b, n, af = atoms.shape
    d = edges.shape[-1]
    bf = bonds.shape[-1]
    hid = W1.shape[-1]
    t = TILE
    e32 = edges.astype(jnp.int32)
    bonds_flat = bonds.reshape(b, n, d * bf)
    w1a, w1b = W1[:af], jnp.tile(W1[af:], (d, 1))
    w2a, w2b = W2[:hid], jnp.tile(W2[hid:], (d, 1))
    woa, wob = Wo[:hid], jnp.tile(Wo[hid:], (d, 1))
    b1r = b1.reshape(1, hid)
    b2r = b2.reshape(1, hid)
    bor = bo.reshape(1, hid)

    grid = (b // t,)
    full = lambda s: pl.BlockSpec(s, lambda i: tuple(0 for _ in s))
    out = pl.pallas_call(
        _fused_body,
        grid=grid,
        in_specs=[
            pl.BlockSpec((t, n, af), lambda i: (i, 0, 0)),
            pl.BlockSpec((t, n, d * bf), lambda i: (i, 0, 0)),
            full(w1a.shape), full(w1b.shape), full(b1r.shape),
            full(w2a.shape), full(w2b.shape), full(b2r.shape),
            full(woa.shape), full(wob.shape), full(bor.shape),
            pl.BlockSpec((t, n, d), lambda i: (i, 0, 0)),
        ],
        out_specs=pl.BlockSpec((t, hid), lambda i: (i, 0)),
        out_shape=jax.ShapeDtypeStruct((b, hid), jnp.float32),
    )(atoms, bonds_flat, w1a, w1b, b1r, w2a, w2b, b2r, woa, wob, bor, e32)
    return out


# speedup vs baseline: 68.6536x; 1.1187x over previous
import jax
import jax.numpy as jnp
from jax.experimental import pallas as pl

TILE = 64


def _fused_body(atoms_ref, bonds_ref, w1a_ref, w1b_ref, b1_ref,
                w2a_ref, w2b_ref, b2_ref, wo_ref, wob_ref, bo_ref,
                edges_ref, out_ref):
    t, n, af = atoms_ref.shape
    d = edges_ref.shape[1]
    hid = w1a_ref.shape[-1]
    t2 = t // 2
    nn = 2 * n

    bonds_flat = bonds_ref[...].reshape(t * n, bonds_ref.shape[-1])
    e3 = edges_ref[...]
    idx = [jnp.broadcast_to(e3[:, k, :][:, None, :], (t2, hid, nn))
           for k in range(d)]

    def gsum_fm(x_fm):
        f = x_fm.shape[1]
        s = x_fm
        for k in range(d):
            s = s + jnp.take_along_axis(x_fm, idx[k][:, :f, :], axis=2,
                                        mode="promise_in_bounds")
        return s

    def gmax_fm(x_fm):
        f = x_fm.shape[1]
        m = x_fm
        for k in range(d):
            m = jnp.maximum(m, jnp.take_along_axis(x_fm, idx[k][:, :f, :],
                                                   axis=2,
                                                   mode="promise_in_bounds"))
        return m

    def to_am(x_fm):
        return jnp.swapaxes(x_fm, 1, 2).reshape(t * n, x_fm.shape[1])

    def to_fm(x_am):
        return jnp.swapaxes(x_am.reshape(t2, nn, x_am.shape[-1]), 1, 2)

    def dense(s_am, wa_ref, wb_ref, b_ref):
        return (jnp.dot(s_am, wa_ref[...], preferred_element_type=jnp.float32)
                + jnp.dot(bonds_flat, wb_ref[...],
                          preferred_element_type=jnp.float32)
                + b_ref[...])

    a_fm = to_fm(atoms_ref[...].reshape(t * n, af))
    s1 = to_am(gsum_fm(a_fm))
    h1 = jnp.maximum(dense(s1, w1a_ref, w1b_ref, b1_ref), 0.0)
    m1_fm = gmax_fm(to_fm(h1))
    s2 = to_am(gsum_fm(m1_fm))
    h2 = jnp.maximum(dense(s2, w2a_ref, w2b_ref, b2_ref), 0.0)
    m2 = to_am(gmax_fm(to_fm(h2)))
    z = dense(m2, wo_ref, wob_ref, bo_ref)
    z = z - jnp.max(z, axis=-1, keepdims=True)
    p = jnp.exp(z)
    p = p / jnp.sum(p, axis=-1, keepdims=True)
    out_ref[...] = jnp.sum(p.reshape(t, n, hid), axis=1)


def kernel(atoms, bonds, W1, b1, W2, b2, Wo, bo, edges):
    b, n, af = atoms.shape
    d = edges.shape[-1]
    bf = bonds.shape[-1]
    hid = W1.shape[-1]
    t = TILE
    afp = (af + 7) // 8 * 8
    b2_ = b // 2
    nn = 2 * n

    atoms_pad = jnp.pad(atoms, ((0, 0), (0, 0), (0, afp - af)))
    bonds_flat = bonds.reshape(b, n, d * bf)
    e_pack = (edges.astype(jnp.int32).reshape(b2_, 2, n, d)
              + jnp.array([0, n], jnp.int32)[None, :, None, None])\
        .transpose(0, 3, 1, 2).reshape(b2_, d, nn)

    w1a = jnp.pad(W1[:af], ((0, afp - af), (0, 0)))
    w1b = jnp.tile(W1[af:], (d, 1))
    w2a, w2b = W2[:hid], jnp.tile(W2[hid:], (d, 1))
    woa, wob = Wo[:hid], jnp.tile(Wo[hid:], (d, 1))
    b1r = b1.reshape(1, hid)
    b2r = b2.reshape(1, hid)
    bor = bo.reshape(1, hid)

    grid = (b // t,)
    full = lambda s: pl.BlockSpec(s, lambda i: tuple(0 for _ in s))
    out = pl.pallas_call(
        _fused_body,
        grid=grid,
        in_specs=[
            pl.BlockSpec((t, n, afp), lambda i: (i, 0, 0)),
            pl.BlockSpec((t, n, d * bf), lambda i: (i, 0, 0)),
            full(w1a.shape), full(w1b.shape), full(b1r.shape),
            full(w2a.shape), full(w2b.shape), full(b2r.shape),
            full(woa.shape), full(wob.shape), full(bor.shape),
            pl.BlockSpec((t // 2, d, nn), lambda i: (i, 0, 0)),
        ],
        out_specs=pl.BlockSpec((t, hid), lambda i: (i, 0)),
        out_shape=jax.ShapeDtypeStruct((b, hid), jnp.float32),
    )(atoms_pad, bonds_flat, w1a, w1b, b1r, w2a, w2b, b2r,
      woa, wob, bor, e_pack)
    return out


# speedup vs baseline: 69.1190x; 1.0068x over previous
import jax
import jax.numpy as jnp
from jax.experimental import pallas as pl

TILE = 128


def _fused_body(atoms_ref, bonds_ref, w1a_ref, w1b_ref, b1_ref,
                w2a_ref, w2b_ref, b2_ref, wo_ref, wob_ref, bo_ref,
                edges_ref, out_ref):
    t, n, af = atoms_ref.shape
    d = edges_ref.shape[1]
    hid = w1a_ref.shape[-1]
    t2 = t // 2
    nn = 2 * n

    bonds_flat = bonds_ref[...].reshape(t * n, bonds_ref.shape[-1])
    e3 = edges_ref[...]
    idx = [jnp.broadcast_to(e3[:, k, :][:, None, :], (t2, hid, nn))
           for k in range(d)]

    def gsum_fm(x_fm):
        f = x_fm.shape[1]
        s = x_fm
        for k in range(d):
            s = s + jnp.take_along_axis(x_fm, idx[k][:, :f, :], axis=2,
                                        mode="promise_in_bounds")
        return s

    def gmax_fm(x_fm):
        f = x_fm.shape[1]
        m = x_fm
        for k in range(d):
            m = jnp.maximum(m, jnp.take_along_axis(x_fm, idx[k][:, :f, :],
                                                   axis=2,
                                                   mode="promise_in_bounds"))
        return m

    def to_am(x_fm):
        return jnp.swapaxes(x_fm, 1, 2).reshape(t * n, x_fm.shape[1])

    def to_fm(x_am):
        return jnp.swapaxes(x_am.reshape(t2, nn, x_am.shape[-1]), 1, 2)

    def dense(s_am, wa_ref, wb_ref, b_ref):
        return (jnp.dot(s_am, wa_ref[...], preferred_element_type=jnp.float32)
                + jnp.dot(bonds_flat, wb_ref[...],
                          preferred_element_type=jnp.float32)
                + b_ref[...])

    a_fm = to_fm(atoms_ref[...].reshape(t * n, af))
    s1 = to_am(gsum_fm(a_fm))
    h1 = jnp.maximum(dense(s1, w1a_ref, w1b_ref, b1_ref), 0.0)
    m1_fm = gmax_fm(to_fm(h1))
    s2 = to_am(gsum_fm(m1_fm))
    h2 = jnp.maximum(dense(s2, w2a_ref, w2b_ref, b2_ref), 0.0)
    m2 = to_am(gmax_fm(to_fm(h2)))
    z = dense(m2, wo_ref, wob_ref, bo_ref)
    z = z - jnp.max(z, axis=-1, keepdims=True)
    p = jnp.exp(z)
    p = p / jnp.sum(p, axis=-1, keepdims=True)
    out_ref[...] = jnp.sum(p.reshape(t, n, hid), axis=1)


def kernel(atoms, bonds, W1, b1, W2, b2, Wo, bo, edges):
    b, n, af = atoms.shape
    d = edges.shape[-1]
    bf = bonds.shape[-1]
    hid = W1.shape[-1]
    t = TILE
    afp = (af + 7) // 8 * 8
    b2_ = b // 2
    nn = 2 * n

    atoms_pad = jnp.pad(atoms, ((0, 0), (0, 0), (0, afp - af)))
    bonds_flat = bonds.reshape(b, n, d * bf)
    e_pack = (edges.astype(jnp.int32).reshape(b2_, 2, n, d)
              + jnp.array([0, n], jnp.int32)[None, :, None, None])\
        .transpose(0, 3, 1, 2).reshape(b2_, d, nn)

    w1a = jnp.pad(W1[:af], ((0, afp - af), (0, 0)))
    w1b = jnp.tile(W1[af:], (d, 1))
    w2a, w2b = W2[:hid], jnp.tile(W2[hid:], (d, 1))
    woa, wob = Wo[:hid], jnp.tile(Wo[hid:], (d, 1))
    b1r = b1.reshape(1, hid)
    b2r = b2.reshape(1, hid)
    bor = bo.reshape(1, hid)

    grid = (b // t,)
    full = lambda s: pl.BlockSpec(s, lambda i: tuple(0 for _ in s))
    out = pl.pallas_call(
        _fused_body,
        grid=grid,
        in_specs=[
            pl.BlockSpec((t, n, afp), lambda i: (i, 0, 0)),
            pl.BlockSpec((t, n, d * bf), lambda i: (i, 0, 0)),
            full(w1a.shape), full(w1b.shape), full(b1r.shape),
            full(w2a.shape), full(w2b.shape), full(b2r.shape),
            full(woa.shape), full(wob.shape), full(bor.shape),
            pl.BlockSpec((t // 2, d, nn), lambda i: (i, 0, 0)),
        ],
        out_specs=pl.BlockSpec((t, hid), lambda i: (i, 0)),
        out_shape=jax.ShapeDtypeStruct((b, hid), jnp.float32),
    )(atoms_pad, bonds_flat, w1a, w1b, b1r, w2a, w2b, b2r,
      woa, wob, bor, e_pack)
    return out


# speedup vs baseline: 69.1212x; 1.0000x over previous
import jax
import jax.numpy as jnp
from jax.experimental import pallas as pl
from jax.experimental.pallas import tpu as pltpu

TILE = 128


def _fused_body(atoms_ref, bonds_ref, w1a_ref, w1b_ref, b1_ref,
                w2a_ref, w2b_ref, b2_ref, wo_ref, wob_ref, bo_ref,
                edges_ref, out_ref):
    t, n, af = atoms_ref.shape
    d = edges_ref.shape[1]
    hid = w1a_ref.shape[-1]
    t2 = t // 2
    nn = 2 * n

    bonds_flat = bonds_ref[...].reshape(t * n, bonds_ref.shape[-1])
    e3 = edges_ref[...]
    idx = [jnp.broadcast_to(e3[:, k, :][:, None, :], (t2, hid, nn))
           for k in range(d)]

    def gsum_fm(x_fm):
        f = x_fm.shape[1]
        s = x_fm
        for k in range(d):
            s = s + jnp.take_along_axis(x_fm, idx[k][:, :f, :], axis=2,
                                        mode="promise_in_bounds")
        return s

    def gmax_fm(x_fm):
        f = x_fm.shape[1]
        m = x_fm
        for k in range(d):
            m = jnp.maximum(m, jnp.take_along_axis(x_fm, idx[k][:, :f, :],
                                                   axis=2,
                                                   mode="promise_in_bounds"))
        return m

    def to_am(x_fm):
        return jnp.swapaxes(x_fm, 1, 2).reshape(t * n, x_fm.shape[1])

    def to_fm(x_am):
        return jnp.swapaxes(x_am.reshape(t2, nn, x_am.shape[-1]), 1, 2)

    def dense(s_am, wa_ref, wb_ref, b_ref):
        return (jnp.dot(s_am, wa_ref[...], preferred_element_type=jnp.float32)
                + jnp.dot(bonds_flat, wb_ref[...],
                          preferred_element_type=jnp.float32)
                + b_ref[...])

    a_fm = to_fm(atoms_ref[...].reshape(t * n, af))
    s1 = to_am(gsum_fm(a_fm))
    h1 = jnp.maximum(dense(s1, w1a_ref, w1b_ref, b1_ref), 0.0)
    m1_fm = gmax_fm(to_fm(h1))
    s2 = to_am(gsum_fm(m1_fm))
    h2 = jnp.maximum(dense(s2, w2a_ref, w2b_ref, b2_ref), 0.0)
    m2 = to_am(gmax_fm(to_fm(h2)))
    z = dense(m2, wo_ref, wob_ref, bo_ref)
    z = z - jnp.max(z, axis=-1, keepdims=True)
    p = jnp.exp(z)
    p = p / jnp.sum(p, axis=-1, keepdims=True)
    out_ref[...] = jnp.sum(p.reshape(t, n, hid), axis=1)


def kernel(atoms, bonds, W1, b1, W2, b2, Wo, bo, edges):
    b, n, af = atoms.shape
    d = edges.shape[-1]
    bf = bonds.shape[-1]
    hid = W1.shape[-1]
    t = TILE
    afp = (af + 7) // 8 * 8
    b2_ = b // 2
    nn = 2 * n

    atoms_pad = jnp.pad(atoms, ((0, 0), (0, 0), (0, afp - af)))
    bonds_flat = bonds.reshape(b, n, d * bf)
    e_pack = (edges.astype(jnp.int32).reshape(b2_, 2, n, d)
              + jnp.array([0, n], jnp.int32)[None, :, None, None])\
        .transpose(0, 3, 1, 2).reshape(b2_, d, nn)

    w1a = jnp.pad(W1[:af], ((0, afp - af), (0, 0)))
    w1b = jnp.tile(W1[af:], (d, 1))
    w2a, w2b = W2[:hid], jnp.tile(W2[hid:], (d, 1))
    woa, wob = Wo[:hid], jnp.tile(Wo[hid:], (d, 1))
    b1r = b1.reshape(1, hid)
    b2r = b2.reshape(1, hid)
    bor = bo.reshape(1, hid)

    grid = (b // t,)
    full = lambda s: pl.BlockSpec(s, lambda i: tuple(0 for _ in s))
    out = pl.pallas_call(
        _fused_body,
        grid=grid,
        in_specs=[
            pl.BlockSpec((t, n, afp), lambda i: (i, 0, 0)),
            pl.BlockSpec((t, n, d * bf), lambda i: (i, 0, 0)),
            full(w1a.shape), full(w1b.shape), full(b1r.shape),
            full(w2a.shape), full(w2b.shape), full(b2r.shape),
            full(woa.shape), full(wob.shape), full(bor.shape),
            pl.BlockSpec((t // 2, d, nn), lambda i: (i, 0, 0)),
        ],
        out_specs=pl.BlockSpec((t, hid), lambda i: (i, 0)),
        out_shape=jax.ShapeDtypeStruct((b, hid), jnp.float32),
        compiler_params=pltpu.CompilerParams(
            dimension_semantics=("parallel",)),
    )(atoms_pad, bonds_flat, w1a, w1b, b1r, w2a, w2b, b2r,
      woa, wob, bor, e_pack)
    return out


# speedup vs baseline: 69.7183x; 1.0086x over previous
import jax
import jax.numpy as jnp
from jax.experimental import pallas as pl
from jax.experimental.pallas import tpu as pltpu

TILE = 128


def _fused_body(atoms_ref, bonds_ref, w1a_ref, w1b_ref, b1_ref,
                w2a_ref, w2b_ref, b2_ref, wo_ref, wob_ref, bo_ref,
                edges_ref, out_ref):
    t, n, af = atoms_ref.shape
    d = edges_ref.shape[1]
    hid = w1a_ref.shape[-1]
    t2 = t // 2
    nn = 2 * n

    bonds_flat = bonds_ref[...].reshape(t * n, bonds_ref.shape[-1])
    e3 = edges_ref[...]
    idx = [jnp.broadcast_to(e3[:, k, :][:, None, :], (t2, hid, nn))
           for k in range(d)]

    def _gathered(x_fm):
        f = x_fm.shape[1]
        return [jnp.take_along_axis(x_fm, idx[k][:, :f, :], axis=2,
                                    mode="promise_in_bounds")
                for k in range(d)]

    def _tree(vals, op):
        while len(vals) > 1:
            vals = [op(vals[i], vals[i + 1]) if i + 1 < len(vals) else vals[i]
                    for i in range(0, len(vals), 2)]
        return vals[0]

    def gsum_fm(x_fm):
        return _tree([x_fm] + _gathered(x_fm), jnp.add)

    def gmax_fm(x_fm):
        return _tree([x_fm] + _gathered(x_fm), jnp.maximum)

    def to_am(x_fm):
        return jnp.swapaxes(x_fm, 1, 2).reshape(t * n, x_fm.shape[1])

    def to_fm(x_am):
        return jnp.swapaxes(x_am.reshape(t2, nn, x_am.shape[-1]), 1, 2)

    def dense(s_am, wa_ref, wb_ref, b_ref):
        return (jnp.dot(s_am, wa_ref[...], preferred_element_type=jnp.float32)
                + jnp.dot(bonds_flat, wb_ref[...],
                          preferred_element_type=jnp.float32)
                + b_ref[...])

    a_fm = to_fm(atoms_ref[...].reshape(t * n, af))
    s1 = to_am(gsum_fm(a_fm))
    h1 = jnp.maximum(dense(s1, w1a_ref, w1b_ref, b1_ref), 0.0)
    m1_fm = gmax_fm(to_fm(h1))
    s2 = to_am(gsum_fm(m1_fm))
    h2 = jnp.maximum(dense(s2, w2a_ref, w2b_ref, b2_ref), 0.0)
    m2 = to_am(gmax_fm(to_fm(h2)))
    z = dense(m2, wo_ref, wob_ref, bo_ref)
    z = z - jnp.max(z, axis=-1, keepdims=True)
    p = jnp.exp(z)
    p = p / jnp.sum(p, axis=-1, keepdims=True)
    out_ref[...] = jnp.sum(p.reshape(t, n, hid), axis=1)


def kernel(atoms, bonds, W1, b1, W2, b2, Wo, bo, edges):
    b, n, af = atoms.shape
    d = edges.shape[-1]
    bf = bonds.shape[-1]
    hid = W1.shape[-1]
    t = TILE
    afp = (af + 7) // 8 * 8
    b2_ = b // 2
    nn = 2 * n

    atoms_pad = jnp.pad(atoms, ((0, 0), (0, 0), (0, afp - af)))
    bonds_flat = bonds.reshape(b, n, d * bf)
    e_pack = (edges.astype(jnp.int32).reshape(b2_, 2, n, d)
              + jnp.array([0, n], jnp.int32)[None, :, None, None])\
        .transpose(0, 3, 1, 2).reshape(b2_, d, nn)

    w1a = jnp.pad(W1[:af], ((0, afp - af), (0, 0)))
    w1b = jnp.tile(W1[af:], (d, 1))
    w2a, w2b = W2[:hid], jnp.tile(W2[hid:], (d, 1))
    woa, wob = Wo[:hid], jnp.tile(Wo[hid:], (d, 1))
    b1r = b1.reshape(1, hid)
    b2r = b2.reshape(1, hid)
    bor = bo.reshape(1, hid)

    grid = (b // t,)
    full = lambda s: pl.BlockSpec(s, lambda i: tuple(0 for _ in s))
    out = pl.pallas_call(
        _fused_body,
        grid=grid,
        in_specs=[
            pl.BlockSpec((t, n, afp), lambda i: (i, 0, 0)),
            pl.BlockSpec((t, n, d * bf), lambda i: (i, 0, 0)),
            full(w1a.shape), full(w1b.shape), full(b1r.shape),
            full(w2a.shape), full(w2b.shape), full(b2r.shape),
            full(woa.shape), full(wob.shape), full(bor.shape),
            pl.BlockSpec((t // 2, d, nn), lambda i: (i, 0, 0)),
        ],
        out_specs=pl.BlockSpec((t, hid), lambda i: (i, 0)),
        out_shape=jax.ShapeDtypeStruct((b, hid), jnp.float32),
        compiler_params=pltpu.CompilerParams(
            dimension_semantics=("parallel",)),
    )(atoms_pad, bonds_flat, w1a, w1b, b1r, w2a, w2b, b2r,
      woa, wob, bor, e_pack)
    return out


# speedup vs baseline: 69.9564x; 1.0034x over previous
import jax
import jax.numpy as jnp
from jax.experimental import pallas as pl
from jax.experimental.pallas import tpu as pltpu

TILE = 128


def _fused_body(atoms_ref, bonds_ref, w1a_ref, w1b_ref, b1_ref,
                w2a_ref, w2b_ref, b2_ref, wo_ref, wob_ref, bo_ref,
                edges_ref, out_ref):
    t, n, af = atoms_ref.shape
    d = edges_ref.shape[1]
    hid = w1a_ref.shape[-1]
    t2 = t // 2
    nn = 2 * n

    bonds_flat = bonds_ref[...].reshape(t * n, bonds_ref.shape[-1])
    e3 = edges_ref[...]
    idx = [jnp.broadcast_to(e3[:, k, :][:, None, :], (t2, hid, nn))
           for k in range(d)]

    def _gathered(x_fm):
        f = x_fm.shape[1]
        return [jnp.take_along_axis(x_fm, idx[k][:, :f, :], axis=2,
                                    mode="promise_in_bounds")
                for k in range(d)]

    def _tree(vals, op):
        while len(vals) > 1:
            vals = [op(vals[i], vals[i + 1]) if i + 1 < len(vals) else vals[i]
                    for i in range(0, len(vals), 2)]
        return vals[0]

    def gsum_fm(x_fm):
        return _tree([x_fm] + _gathered(x_fm), jnp.add)

    def gmax_fm(x_fm):
        return _tree([x_fm] + _gathered(x_fm), jnp.maximum)

    def to_am(x_fm):
        return jnp.swapaxes(x_fm, 1, 2).reshape(t * n, x_fm.shape[1])

    def to_fm(x_am):
        return jnp.swapaxes(x_am.reshape(t2, nn, x_am.shape[-1]), 1, 2)

    def dense(s_am, wa_ref, zb, b_ref):
        return (jnp.dot(s_am, wa_ref[...], preferred_element_type=jnp.float32)
                + zb + b_ref[...])

    zb1 = jnp.dot(bonds_flat, w1b_ref[...], preferred_element_type=jnp.float32)
    zb2 = jnp.dot(bonds_flat, w2b_ref[...], preferred_element_type=jnp.float32)
    zbo = jnp.dot(bonds_flat, wob_ref[...], preferred_element_type=jnp.float32)

    a_fm = to_fm(atoms_ref[...].reshape(t * n, af))
    s1 = to_am(gsum_fm(a_fm))
    h1 = jnp.maximum(dense(s1, w1a_ref, zb1, b1_ref), 0.0)
    m1_fm = gmax_fm(to_fm(h1))
    s2 = to_am(gsum_fm(m1_fm))
    h2 = jnp.maximum(dense(s2, w2a_ref, zb2, b2_ref), 0.0)
    m2 = to_am(gmax_fm(to_fm(h2)))
    z = dense(m2, wo_ref, zbo, bo_ref)
    z = z - jnp.max(z, axis=-1, keepdims=True)
    p = jnp.exp(z)
    p = p / jnp.sum(p, axis=-1, keepdims=True)
    out_ref[...] = jnp.sum(p.reshape(t, n, hid), axis=1)


def kernel(atoms, bonds, W1, b1, W2, b2, Wo, bo, edges):
    b, n, af = atoms.shape
    d = edges.shape[-1]
    bf = bonds.shape[-1]
    hid = W1.shape[-1]
    t = TILE
    afp = (af + 7) // 8 * 8
    b2_ = b // 2
    nn = 2 * n

    atoms_pad = jnp.pad(atoms, ((0, 0), (0, 0), (0, afp - af)))
    bonds_flat = bonds.reshape(b, n, d * bf)
    e_pack = (edges.astype(jnp.int32).reshape(b2_, 2, n, d)
              + jnp.array([0, n], jnp.int32)[None, :, None, None])\
        .transpose(0, 3, 1, 2).reshape(b2_, d, nn)

    w1a = jnp.pad(W1[:af], ((0, afp - af), (0, 0)))
    w1b = jnp.tile(W1[af:], (d, 1))
    w2a, w2b = W2[:hid], jnp.tile(W2[hid:], (d, 1))
    woa, wob = Wo[:hid], jnp.tile(Wo[hid:], (d, 1))
    b1r = b1.reshape(1, hid)
    b2r = b2.reshape(1, hid)
    bor = bo.reshape(1, hid)

    grid = (b // t,)
    full = lambda s: pl.BlockSpec(s, lambda i: tuple(0 for _ in s))
    out = pl.pallas_call(
        _fused_body,
        grid=grid,
        in_specs=[
            pl.BlockSpec((t, n, afp), lambda i: (i, 0, 0)),
            pl.BlockSpec((t, n, d * bf), lambda i: (i, 0, 0)),
            full(w1a.shape), full(w1b.shape), full(b1r.shape),
            full(w2a.shape), full(w2b.shape), full(b2r.shape),
            full(woa.shape), full(wob.shape), full(bor.shape),
            pl.BlockSpec((t // 2, d, nn), lambda i: (i, 0, 0)),
        ],
        out_specs=pl.BlockSpec((t, hid), lambda i: (i, 0)),
        out_shape=jax.ShapeDtypeStruct((b, hid), jnp.float32),
        compiler_params=pltpu.CompilerParams(
            dimension_semantics=("parallel",)),
    )(atoms_pad, bonds_flat, w1a, w1b, b1r, w2a, w2b, b2r,
      woa, wob, bor, e_pack)
    return out
